# baseline (speedup 1.0000x reference)
import functools

import jax
import jax.numpy as jnp
from jax.experimental import pallas as pl
from jax.experimental.pallas import tpu as pltpu

_PAD = 128

_CP = pltpu.CompilerParams(
    dimension_semantics=("parallel",),
    vmem_limit_bytes=60 * 1024 * 1024,
)


def _taps(w):
    co, ci, kh, kw = w.shape
    t = jnp.transpose(w, (2, 3, 0, 1)).reshape(kh * kw, co, ci)
    return t.astype(jnp.bfloat16)


def _flat(x):
    b, c, h, w = x.shape
    f = jnp.transpose(x, (1, 0, 2, 3)).reshape(c, b * h * w)
    return jnp.pad(f.astype(jnp.bfloat16), ((0, 0), (_PAD, _PAD)))


def _tap_slice(wide, tm, dh, dw, hrow, wcol, h_, w_):
    d = _PAD + dh * w_ + dw
    sl = jax.lax.slice_in_dim(wide, d, d + tm, axis=1)
    cond = None
    if dh < 0:
        cond = hrow > 0
    elif dh > 0:
        cond = hrow < h_ - 1
    if dw < 0:
        c2 = wcol > 0
        cond = c2 if cond is None else (cond & c2)
    elif dw > 0:
        c2 = wcol < w_ - 1
        cond = c2 if cond is None else (cond & c2)
    if cond is not None:
        sl = jnp.where(cond, sl, jnp.zeros_like(sl))
    return sl


def _hi_kernel(wt_ref, x_ref, u_ref, y_ref, s_ref, q_ref, *, tm, h_, w_):
    m0 = pl.program_id(0) * tm
    idx = jax.lax.broadcasted_iota(jnp.int32, (1, tm), 1) + m0
    wcol = idx & (w_ - 1)
    hrow = (idx // w_) & (h_ - 1)
    acc = u_ref[...].astype(jnp.float32)
    wide = x_ref[:, pl.ds(m0, tm + 2 * _PAD)]
    for t in range(9):
        dh, dw = t // 3 - 1, t % 3 - 1
        sl = _tap_slice(wide, tm, dh, dw, hrow, wcol, h_, w_)
        acc = acc + jnp.dot(wt_ref[t], sl, preferred_element_type=jnp.float32)
    y_ref[...] = acc.astype(y_ref.dtype)
    s_ref[...] = jnp.sum(acc, axis=1, keepdims=True)[None]
    q_ref[...] = jnp.sum(acc * acc, axis=1, keepdims=True)[None]


def _lo_kernel(wh2l_ref, wl2l_ref, wl2h_ref, p_ref, x_ref,
               ylo_ref, yl2h_ref, s_ref, q_ref, *, tm, h_, w_):
    m0 = pl.program_id(0) * tm
    idx = jax.lax.broadcasted_iota(jnp.int32, (1, tm), 1) + m0
    wcol = idx & (w_ - 1)
    hrow = (idx // w_) & (h_ - 1)
    acc = jnp.zeros(ylo_ref.shape, jnp.float32)
    acc2 = jnp.zeros(yl2h_ref.shape, jnp.float32)
    wide_p = p_ref[:, pl.ds(m0, tm + 2 * _PAD)]
    wide_x = x_ref[:, pl.ds(m0, tm + 2 * _PAD)]
    for t in range(9):
        dh, dw = t // 3 - 1, t % 3 - 1
        sp = _tap_slice(wide_p, tm, dh, dw, hrow, wcol, h_, w_)
        sx = _tap_slice(wide_x, tm, dh, dw, hrow, wcol, h_, w_)
        acc = (acc + jnp.dot(wh2l_ref[t], sp, preferred_element_type=jnp.float32)
               + jnp.dot(wl2l_ref[t], sx, preferred_element_type=jnp.float32))
        acc2 = acc2 + jnp.dot(wl2h_ref[t], sx, preferred_element_type=jnp.float32)
    ylo_ref[...] = acc.astype(ylo_ref.dtype)
    yl2h_ref[...] = acc2.astype(yl2h_ref.dtype)
    s_ref[...] = jnp.sum(acc, axis=1, keepdims=True)[None]
    q_ref[...] = jnp.sum(acc * acc, axis=1, keepdims=True)[None]


def _affine_kernel(y_ref, scale_ref, shift_ref, o_ref):
    o_ref[...] = (y_ref[...].astype(jnp.float32) * scale_ref[...]
                  + shift_ref[...]).astype(o_ref.dtype)


def _bn_apply(y, part_s, part_q, m, gamma, beta, eps, tm):
    c, mp = y.shape
    mean = jnp.sum(part_s, axis=0)[:, 0] / m
    var = jnp.sum(part_q, axis=0)[:, 0] / m - mean * mean
    scale = gamma.astype(jnp.float32) * jax.lax.rsqrt(var + eps)
    shift = beta.astype(jnp.float32) - mean * scale
    n = mp // tm
    return pl.pallas_call(
        _affine_kernel,
        out_shape=jax.ShapeDtypeStruct((c, mp), jnp.float32),
        grid=(n,),
        in_specs=[
            pl.BlockSpec((c, tm), lambda i: (0, i)),
            pl.BlockSpec((c, 1), lambda i: (0, 0)),
            pl.BlockSpec((c, 1), lambda i: (0, 0)),
        ],
        out_specs=pl.BlockSpec((c, tm), lambda i: (0, i)),
        compiler_params=_CP,
    )(y, scale[:, None], shift[:, None])


def kernel(w_h2h, w_h2l, w_l2h, w_l2l, gamma_h, beta_h, gamma_l, beta_l,
           x_h, x_l, eps=1e-5):
    b, cih, h, w = x_h.shape
    _, cil, hl, wl = x_l.shape
    coh = w_h2h.shape[0]
    col = w_l2l.shape[0]
    assert h & (h - 1) == 0 and w & (w - 1) == 0, "spatial dims must be pow2"
    mh, ml = b * h * w, b * hl * wl

    xh_f = _flat(x_h)
    pool = x_h.reshape(b, cih, hl, 2, wl, 2).mean(axis=(3, 5))
    ph_f = _flat(pool)
    xl_f = _flat(x_l)

    tml = min(512, ml)
    nl = ml // tml
    kl = functools.partial(_lo_kernel, tm=tml, h_=hl, w_=wl)
    mlp2 = ml + 2 * _PAD
    y_lo, y_l2h, s_lo, q_lo = pl.pallas_call(
        kl,
        out_shape=(jax.ShapeDtypeStruct((col, ml), jnp.float32),
                   jax.ShapeDtypeStruct((coh, ml), jnp.float32),
                   jax.ShapeDtypeStruct((nl, col, 1), jnp.float32),
                   jax.ShapeDtypeStruct((nl, col, 1), jnp.float32)),
        grid=(nl,),
        in_specs=[
            pl.BlockSpec((9, col, cih), lambda i: (0, 0, 0)),
            pl.BlockSpec((9, col, cil), lambda i: (0, 0, 0)),
            pl.BlockSpec((9, coh, cil), lambda i: (0, 0, 0)),
            pl.BlockSpec((cih, mlp2), lambda i: (0, 0)),
            pl.BlockSpec((cil, mlp2), lambda i: (0, 0)),
        ],
        out_specs=(pl.BlockSpec((col, tml), lambda i: (0, i)),
                   pl.BlockSpec((coh, tml), lambda i: (0, i)),
                   pl.BlockSpec((1, col, 1), lambda i: (i, 0, 0)),
                   pl.BlockSpec((1, col, 1), lambda i: (i, 0, 0))),
        compiler_params=_CP,
    )(_taps(w_h2l), _taps(w_l2l), _taps(w_l2h), ph_f, xl_f)

    u = jnp.repeat(jnp.repeat(y_l2h.reshape(coh, b, hl, wl), 2, axis=2),
                   2, axis=3).reshape(coh, mh)

    tmh = min(512, mh)
    nh = mh // tmh
    kh = functools.partial(_hi_kernel, tm=tmh, h_=h, w_=w)
    mhp2 = mh + 2 * _PAD
    y_hi, s_hi, q_hi = pl.pallas_call(
        kh,
        out_shape=(jax.ShapeDtypeStruct((coh, mh), jnp.float32),
                   jax.ShapeDtypeStruct((nh, coh, 1), jnp.float32),
                   jax.ShapeDtypeStruct((nh, coh, 1), jnp.float32)),
        grid=(nh,),
        in_specs=[
            pl.BlockSpec((9, coh, cih), lambda i: (0, 0, 0)),
            pl.BlockSpec((cih, mhp2), lambda i: (0, 0)),
            pl.BlockSpec((coh, tmh), lambda i: (0, i)),
        ],
        out_specs=(pl.BlockSpec((coh, tmh), lambda i: (0, i)),
                   pl.BlockSpec((1, coh, 1), lambda i: (i, 0, 0)),
                   pl.BlockSpec((1, coh, 1), lambda i: (i, 0, 0))),
        compiler_params=_CP,
    )(_taps(w_h2h), xh_f, u)

    out_h2 = _bn_apply(y_hi, s_hi, q_hi, mh, gamma_h, beta_h, eps, tmh)
    out_l2 = _bn_apply(y_lo, s_lo, q_lo, ml, gamma_l, beta_l, eps, tml)

    out_h = jnp.transpose(out_h2.reshape(coh, b, h, w), (1, 0, 2, 3))
    out_l = jnp.transpose(out_l2.reshape(col, b, hl, wl), (1, 0, 2, 3))
    return out_h, out_l


# speedup vs baseline: 1.4876x; 1.4876x over previous
import functools

import jax
import jax.numpy as jnp
from jax.experimental import pallas as pl
from jax.experimental.pallas import tpu as pltpu

_CP = pltpu.CompilerParams(
    dimension_semantics=("parallel",),
    vmem_limit_bytes=60 * 1024 * 1024,
)


def _taps(w):
    co, ci, kh, kw = w.shape
    t = jnp.transpose(w, (2, 3, 0, 1)).reshape(kh * kw, co, ci)
    return t.astype(jnp.bfloat16)


def _conv9(wt_ref, xp, acc, tm, dpad, w_, hr, wc, h_):
    for t in range(9):
        dh, dw = t // 3 - 1, t % 3 - 1
        d = dpad + dh * w_ + dw
        sl = jax.lax.slice_in_dim(xp, d, d + tm, axis=1)
        cond = None
        if dh < 0:
            cond = hr > 0
        elif dh > 0:
            cond = hr < h_ - 1
        if dw < 0:
            c2 = wc > 0
            cond = c2 if cond is None else (cond & c2)
        elif dw > 0:
            c2 = wc < w_ - 1
            cond = c2 if cond is None else (cond & c2)
        if cond is not None:
            sl = jnp.where(cond, sl, jnp.zeros_like(sl))
        acc = acc + jnp.dot(wt_ref[t], sl, preferred_element_type=jnp.float32)
    return acc


def _fused_kernel(wh2h_ref, wh2l_ref, wl2l_ref, wl2h_ref, pool_ref, up_ref,
                  xh_ref, xl_ref,
                  yhi_ref, ylo_ref, sh_ref, qh_ref, sl_ref, ql_ref,
                  *, h, w, hl, wl):
    hw, hwl = h * w, hl * wl
    xh = xh_ref[0].astype(jnp.bfloat16)
    xl = xl_ref[0].astype(jnp.bfloat16)
    pooled = jnp.dot(xh, pool_ref[...],
                     preferred_element_type=jnp.float32)
    pooled = pooled.astype(jnp.bfloat16)

    idx_l = jax.lax.broadcasted_iota(jnp.int32, (1, hwl), 1)
    wc_l = idx_l & (wl - 1)
    hr_l = idx_l >> (wl.bit_length() - 1)
    idx_h = jax.lax.broadcasted_iota(jnp.int32, (1, hw), 1)
    wc_h = idx_h & (w - 1)
    hr_h = idx_h >> (w.bit_length() - 1)

    pl_pad = wl + 1
    xlp = jnp.pad(xl, ((0, 0), (pl_pad, pl_pad)))
    plp = jnp.pad(pooled, ((0, 0), (pl_pad, pl_pad)))
    ph_pad = w + 1
    xhp = jnp.pad(xh, ((0, 0), (ph_pad, ph_pad)))

    col = ylo_ref.shape[1]
    coh = yhi_ref.shape[1]
    acc_lo = jnp.zeros((col, hwl), jnp.float32)
    acc_lo = _conv9(wh2l_ref, plp, acc_lo, hwl, pl_pad, wl, hr_l, wc_l, hl)
    acc_lo = _conv9(wl2l_ref, xlp, acc_lo, hwl, pl_pad, wl, hr_l, wc_l, hl)
    acc_l2h = jnp.zeros((coh, hwl), jnp.float32)
    acc_l2h = _conv9(wl2h_ref, xlp, acc_l2h, hwl, pl_pad, wl, hr_l, wc_l, hl)

    acc_hi = jnp.dot(acc_l2h.astype(jnp.bfloat16), up_ref[...],
                     preferred_element_type=jnp.float32)
    acc_hi = _conv9(wh2h_ref, xhp, acc_hi, hw, ph_pad, w, hr_h, wc_h, h)

    yhi_ref[...] = acc_hi.astype(yhi_ref.dtype)[None]
    ylo_ref[...] = acc_lo.astype(ylo_ref.dtype)[None]
    sh_ref[...] = jnp.sum(acc_hi, axis=1, keepdims=True)[None]
    qh_ref[...] = jnp.sum(acc_hi * acc_hi, axis=1, keepdims=True)[None]
    sl_ref[...] = jnp.sum(acc_lo, axis=1, keepdims=True)[None]
    ql_ref[...] = jnp.sum(acc_lo * acc_lo, axis=1, keepdims=True)[None]


def _affine_kernel(y_ref, scale_ref, shift_ref, o_ref):
    o_ref[...] = (y_ref[...].astype(jnp.float32) * scale_ref[...]
                  + shift_ref[...]).astype(o_ref.dtype)


def _bn_apply(y, part_s, part_q, m, gamma, beta, eps):
    b, c, hw = y.shape
    mean = jnp.sum(part_s, axis=0)[:, 0] / m
    var = jnp.sum(part_q, axis=0)[:, 0] / m - mean * mean
    scale = gamma.astype(jnp.float32) * jax.lax.rsqrt(var + eps)
    shift = beta.astype(jnp.float32) - mean * scale
    nb = 2 if b % 2 == 0 else 1
    return pl.pallas_call(
        _affine_kernel,
        out_shape=jax.ShapeDtypeStruct((b, c, hw), jnp.float32),
        grid=(b // nb,),
        in_specs=[
            pl.BlockSpec((nb, c, hw), lambda i: (i, 0, 0)),
            pl.BlockSpec((c, 1), lambda i: (0, 0)),
            pl.BlockSpec((c, 1), lambda i: (0, 0)),
        ],
        out_specs=pl.BlockSpec((nb, c, hw), lambda i: (i, 0, 0)),
        compiler_params=_CP,
    )(y, scale[:, None], shift[:, None])


def kernel(w_h2h, w_h2l, w_l2h, w_l2l, gamma_h, beta_h, gamma_l, beta_l,
           x_h, x_l, eps=1e-5):
    b, cih, h, w = x_h.shape
    _, cil, hl, wl = x_l.shape
    coh = w_h2h.shape[0]
    col = w_l2l.shape[0]
    assert h & (h - 1) == 0 and w & (w - 1) == 0, "spatial dims must be pow2"
    hw, hwl = h * w, hl * wl
    mh, ml = b * hw, b * hwl

    q = jnp.arange(hw)
    p_of_q = (q // (2 * w)) * wl + (q % w) // 2
    sel = p_of_q[:, None] == jnp.arange(hwl)[None, :]
    pool_mat = jnp.where(sel, 0.25, 0.0).astype(jnp.bfloat16)
    up_mat = jnp.where(sel, 1.0, 0.0).astype(jnp.bfloat16).T

    kfn = functools.partial(_fused_kernel, h=h, w=w, hl=hl, wl=wl)
    y_hi, y_lo, s_hi, q_hi, s_lo, q_lo = pl.pallas_call(
        kfn,
        out_shape=(jax.ShapeDtypeStruct((b, coh, hw), jnp.bfloat16),
                   jax.ShapeDtypeStruct((b, col, hwl), jnp.bfloat16),
                   jax.ShapeDtypeStruct((b, coh, 1), jnp.float32),
                   jax.ShapeDtypeStruct((b, coh, 1), jnp.float32),
                   jax.ShapeDtypeStruct((b, col, 1), jnp.float32),
                   jax.ShapeDtypeStruct((b, col, 1), jnp.float32)),
        grid=(b,),
        in_specs=[
            pl.BlockSpec((9, coh, cih), lambda i: (0, 0, 0)),
            pl.BlockSpec((9, col, cih), lambda i: (0, 0, 0)),
            pl.BlockSpec((9, col, cil), lambda i: (0, 0, 0)),
            pl.BlockSpec((9, coh, cil), lambda i: (0, 0, 0)),
            pl.BlockSpec((hw, hwl), lambda i: (0, 0)),
            pl.BlockSpec((hwl, hw), lambda i: (0, 0)),
            pl.BlockSpec((1, cih, hw), lambda i: (i, 0, 0)),
            pl.BlockSpec((1, cil, hwl), lambda i: (i, 0, 0)),
        ],
        out_specs=(pl.BlockSpec((1, coh, hw), lambda i: (i, 0, 0)),
                   pl.BlockSpec((1, col, hwl), lambda i: (i, 0, 0)),
                   pl.BlockSpec((1, coh, 1), lambda i: (i, 0, 0)),
                   pl.BlockSpec((1, coh, 1), lambda i: (i, 0, 0)),
                   pl.BlockSpec((1, col, 1), lambda i: (i, 0, 0)),
                   pl.BlockSpec((1, col, 1), lambda i: (i, 0, 0))),
        compiler_params=_CP,
    )(_taps(w_h2h), _taps(w_h2l), _taps(w_l2l), _taps(w_l2h),
      pool_mat, up_mat, x_h.reshape(b, cih, hw), x_l.reshape(b, cil, hwl))

    out_h = _bn_apply(y_hi, s_hi, q_hi, mh, gamma_h, beta_h, eps)
    out_l = _bn_apply(y_lo, s_lo, q_lo, ml, gamma_l, beta_l, eps)
    return out_h.reshape(b, coh, h, w), out_l.reshape(b, col, hl, wl)


# speedup vs baseline: 1.9112x; 1.2847x over previous
import functools

import jax
import jax.numpy as jnp
from jax.experimental import pallas as pl
from jax.experimental.pallas import tpu as pltpu

_CP = pltpu.CompilerParams(
    dimension_semantics=("parallel",),
    vmem_limit_bytes=60 * 1024 * 1024,
)


def _taps(w):
    co, ci, kh, kw = w.shape
    t = jnp.transpose(w, (2, 3, 0, 1)).reshape(kh * kw, co, ci)
    return t.astype(jnp.bfloat16)


def _conv9(wt_ref, xp, acc, tm, dpad, w_, hr, wc, h_):
    for t in range(9):
        dh, dw = t // 3 - 1, t % 3 - 1
        d = dpad + dh * w_ + dw
        sl = jax.lax.slice_in_dim(xp, d, d + tm, axis=1)
        cond = None
        if dh < 0:
            cond = hr > 0
        elif dh > 0:
            cond = hr < h_ - 1
        if dw < 0:
            c2 = wc > 0
            cond = c2 if cond is None else (cond & c2)
        elif dw > 0:
            c2 = wc < w_ - 1
            cond = c2 if cond is None else (cond & c2)
        if cond is not None:
            sl = jnp.where(cond, sl, jnp.zeros_like(sl))
        acc = acc + jnp.dot(wt_ref[t], sl, preferred_element_type=jnp.float32)
    return acc


def _fused_kernel(wh2h_ref, wh2l_ref, wl2l_ref, wl2h_ref, pool_ref, up_ref,
                  xh_ref, xl_ref,
                  yhi_ref, ylo_ref, ph_ref, plo_ref,
                  *, h, w, hl, wl, nb):
    hw, hwl = h * w, hl * wl
    idx_l = jax.lax.broadcasted_iota(jnp.int32, (1, hwl), 1)
    wc_l = idx_l & (wl - 1)
    hr_l = idx_l >> (wl.bit_length() - 1)
    idx_h = jax.lax.broadcasted_iota(jnp.int32, (1, hw), 1)
    wc_h = idx_h & (w - 1)
    hr_h = idx_h >> (w.bit_length() - 1)
    pl_pad = wl + 1
    ph_pad = w + 1
    col = ylo_ref.shape[1]
    coh = yhi_ref.shape[1]

    for j in range(nb):
        xh = xh_ref[j].astype(jnp.bfloat16)
        xl = xl_ref[j].astype(jnp.bfloat16)
        pooled = jnp.dot(xh, pool_ref[...],
                         preferred_element_type=jnp.float32)
        pooled = pooled.astype(jnp.bfloat16)

        xlp = jnp.pad(xl, ((0, 0), (pl_pad, pl_pad)))
        plp = jnp.pad(pooled, ((0, 0), (pl_pad, pl_pad)))
        xhp = jnp.pad(xh, ((0, 0), (ph_pad, ph_pad)))

        acc_lo = jnp.zeros((col, hwl), jnp.float32)
        acc_lo = _conv9(wh2l_ref, plp, acc_lo, hwl, pl_pad, wl, hr_l, wc_l, hl)
        acc_lo = _conv9(wl2l_ref, xlp, acc_lo, hwl, pl_pad, wl, hr_l, wc_l, hl)
        acc_l2h = jnp.zeros((coh, hwl), jnp.float32)
        acc_l2h = _conv9(wl2h_ref, xlp, acc_l2h, hwl, pl_pad, wl, hr_l, wc_l, hl)

        acc_hi = jnp.dot(acc_l2h.astype(jnp.bfloat16), up_ref[...],
                         preferred_element_type=jnp.float32)
        acc_hi = _conv9(wh2h_ref, xhp, acc_hi, hw, ph_pad, w, hr_h, wc_h, h)

        yhi_ref[j] = acc_hi.astype(yhi_ref.dtype)
        ylo_ref[j] = acc_lo.astype(ylo_ref.dtype)
        ph_ref[j] = jnp.concatenate(
            [jnp.sum(acc_hi, axis=1, keepdims=True),
             jnp.sum(acc_hi * acc_hi, axis=1, keepdims=True)], axis=1)
        plo_ref[j] = jnp.concatenate(
            [jnp.sum(acc_lo, axis=1, keepdims=True),
             jnp.sum(acc_lo * acc_lo, axis=1, keepdims=True)], axis=1)


def _affine_kernel(y_ref, scale_ref, shift_ref, o_ref):
    o_ref[...] = (y_ref[...].astype(jnp.float32) * scale_ref[...]
                  + shift_ref[...]).astype(o_ref.dtype)


def _bn_apply(y, part, m, gamma, beta, eps):
    b, c, hw = y.shape
    mean = jnp.sum(part[:, :, 0], axis=0) / m
    var = jnp.sum(part[:, :, 1], axis=0) / m - mean * mean
    scale = gamma.astype(jnp.float32) * jax.lax.rsqrt(var + eps)
    shift = beta.astype(jnp.float32) - mean * scale
    nb = 2 if b % 2 == 0 else 1
    return pl.pallas_call(
        _affine_kernel,
        out_shape=jax.ShapeDtypeStruct((b, c, hw), jnp.float32),
        grid=(b // nb,),
        in_specs=[
            pl.BlockSpec((nb, c, hw), lambda i: (i, 0, 0)),
            pl.BlockSpec((c, 1), lambda i: (0, 0)),
            pl.BlockSpec((c, 1), lambda i: (0, 0)),
        ],
        out_specs=pl.BlockSpec((nb, c, hw), lambda i: (i, 0, 0)),
        compiler_params=_CP,
    )(y, scale[:, None], shift[:, None])


def kernel(w_h2h, w_h2l, w_l2h, w_l2l, gamma_h, beta_h, gamma_l, beta_l,
           x_h, x_l, eps=1e-5):
    b, cih, h, w = x_h.shape
    _, cil, hl, wl = x_l.shape
    coh = w_h2h.shape[0]
    col = w_l2l.shape[0]
    assert h & (h - 1) == 0 and w & (w - 1) == 0, "spatial dims must be pow2"
    hw, hwl = h * w, hl * wl
    mh, ml = b * hw, b * hwl

    q = jnp.arange(hw)
    p_of_q = (q // (2 * w)) * wl + (q % w) // 2
    sel = p_of_q[:, None] == jnp.arange(hwl)[None, :]
    pool_mat = jnp.where(sel, 0.25, 0.0).astype(jnp.bfloat16)
    up_mat = jnp.where(sel, 1.0, 0.0).astype(jnp.bfloat16).T

    nb = 4 if b % 4 == 0 else 1
    kfn = functools.partial(_fused_kernel, h=h, w=w, hl=hl, wl=wl, nb=nb)
    y_hi, y_lo, p_hi, p_lo = pl.pallas_call(
        kfn,
        out_shape=(jax.ShapeDtypeStruct((b, coh, hw), jnp.bfloat16),
                   jax.ShapeDtypeStruct((b, col, hwl), jnp.bfloat16),
                   jax.ShapeDtypeStruct((b, coh, 2), jnp.float32),
                   jax.ShapeDtypeStruct((b, col, 2), jnp.float32)),
        grid=(b // nb,),
        in_specs=[
            pl.BlockSpec((9, coh, cih), lambda i: (0, 0, 0)),
            pl.BlockSpec((9, col, cih), lambda i: (0, 0, 0)),
            pl.BlockSpec((9, col, cil), lambda i: (0, 0, 0)),
            pl.BlockSpec((9, coh, cil), lambda i: (0, 0, 0)),
            pl.BlockSpec((hw, hwl), lambda i: (0, 0)),
            pl.BlockSpec((hwl, hw), lambda i: (0, 0)),
            pl.BlockSpec((nb, cih, hw), lambda i: (i, 0, 0)),
            pl.BlockSpec((nb, cil, hwl), lambda i: (i, 0, 0)),
        ],
        out_specs=(pl.BlockSpec((nb, coh, hw), lambda i: (i, 0, 0)),
                   pl.BlockSpec((nb, col, hwl), lambda i: (i, 0, 0)),
                   pl.BlockSpec((nb, coh, 2), lambda i: (i, 0, 0)),
                   pl.BlockSpec((nb, col, 2), lambda i: (i, 0, 0))),
        compiler_params=_CP,
    )(_taps(w_h2h), _taps(w_h2l), _taps(w_l2l), _taps(w_l2h),
      pool_mat, up_mat, x_h.reshape(b, cih, hw), x_l.reshape(b, cil, hwl))

    out_h = _bn_apply(y_hi, p_hi, mh, gamma_h, beta_h, eps)
    out_l = _bn_apply(y_lo, p_lo, ml, gamma_l, beta_l, eps)
    return out_h.reshape(b, coh, h, w), out_l.reshape(b, col, hl, wl)


# speedup vs baseline: 1.9159x; 1.0025x over previous
import functools

import jax
import jax.numpy as jnp
from jax.experimental import pallas as pl
from jax.experimental.pallas import tpu as pltpu

_CP = pltpu.CompilerParams(
    dimension_semantics=("parallel",),
    vmem_limit_bytes=60 * 1024 * 1024,
)


def _taps(w):
    co, ci, kh, kw = w.shape
    t = jnp.transpose(w, (2, 3, 0, 1)).reshape(kh * kw, co, ci)
    return t.astype(jnp.bfloat16)


def _conv9(wt_ref, xp, acc, tm, dpad, w_, hr, wc, h_):
    for t in range(9):
        dh, dw = t // 3 - 1, t % 3 - 1
        d = dpad + dh * w_ + dw
        sl = jax.lax.slice_in_dim(xp, d, d + tm, axis=1)
        cond = None
        if dh < 0:
            cond = hr > 0
        elif dh > 0:
            cond = hr < h_ - 1
        if dw < 0:
            c2 = wc > 0
            cond = c2 if cond is None else (cond & c2)
        elif dw > 0:
            c2 = wc < w_ - 1
            cond = c2 if cond is None else (cond & c2)
        if cond is not None:
            sl = jnp.where(cond, sl, jnp.zeros_like(sl))
        acc = acc + jnp.dot(wt_ref[t], sl, preferred_element_type=jnp.float32)
    return acc


def _fused_kernel(wh2h_ref, wh2l_ref, wl2l_ref, wl2h_ref, pool_ref, up_ref,
                  xh_ref, xl_ref,
                  yhi_ref, ylo_ref, ph_ref, plo_ref,
                  *, h, w, hl, wl, nb):
    hw, hwl = h * w, hl * wl
    idx_l = jax.lax.broadcasted_iota(jnp.int32, (1, hwl), 1)
    wc_l = idx_l & (wl - 1)
    hr_l = idx_l >> (wl.bit_length() - 1)
    idx_h = jax.lax.broadcasted_iota(jnp.int32, (1, hw), 1)
    wc_h = idx_h & (w - 1)
    hr_h = idx_h >> (w.bit_length() - 1)
    pl_pad = wl + 1
    ph_pad = w + 1
    col = ylo_ref.shape[1]
    coh = yhi_ref.shape[1]

    for j in range(nb):
        xh = xh_ref[j].astype(jnp.bfloat16)
        xl = xl_ref[j].astype(jnp.bfloat16)
        pooled = jnp.dot(xh, pool_ref[...],
                         preferred_element_type=jnp.float32)
        pooled = pooled.astype(jnp.bfloat16)

        xlp = jnp.pad(xl, ((0, 0), (pl_pad, pl_pad)))
        plp = jnp.pad(pooled, ((0, 0), (pl_pad, pl_pad)))
        xhp = jnp.pad(xh, ((0, 0), (ph_pad, ph_pad)))

        acc_lo = jnp.zeros((col, hwl), jnp.float32)
        acc_lo = _conv9(wh2l_ref, plp, acc_lo, hwl, pl_pad, wl, hr_l, wc_l, hl)
        acc_lo = _conv9(wl2l_ref, xlp, acc_lo, hwl, pl_pad, wl, hr_l, wc_l, hl)
        acc_l2h = jnp.zeros((coh, hwl), jnp.float32)
        acc_l2h = _conv9(wl2h_ref, xlp, acc_l2h, hwl, pl_pad, wl, hr_l, wc_l, hl)

        acc_hi = jnp.dot(acc_l2h.astype(jnp.bfloat16), up_ref[...],
                         preferred_element_type=jnp.float32)
        acc_hi = _conv9(wh2h_ref, xhp, acc_hi, hw, ph_pad, w, hr_h, wc_h, h)

        yhi_ref[j] = acc_hi.astype(yhi_ref.dtype)
        ylo_ref[j] = acc_lo.astype(ylo_ref.dtype)
        ph_ref[j] = jnp.concatenate(
            [jnp.sum(acc_hi, axis=1, keepdims=True),
             jnp.sum(acc_hi * acc_hi, axis=1, keepdims=True)], axis=1)
        plo_ref[j] = jnp.concatenate(
            [jnp.sum(acc_lo, axis=1, keepdims=True),
             jnp.sum(acc_lo * acc_lo, axis=1, keepdims=True)], axis=1)


def _affine_kernel(y_ref, scale_ref, shift_ref, o_ref):
    o_ref[...] = (y_ref[...].astype(jnp.float32) * scale_ref[...]
                  + shift_ref[...]).astype(o_ref.dtype)


def _bn_apply(y, part, m, gamma, beta, eps):
    b, c, hw = y.shape
    mean = jnp.sum(part[:, :, 0], axis=0) / m
    var = jnp.sum(part[:, :, 1], axis=0) / m - mean * mean
    scale = gamma.astype(jnp.float32) * jax.lax.rsqrt(var + eps)
    shift = beta.astype(jnp.float32) - mean * scale
    nb = 2 if b % 2 == 0 else 1
    return pl.pallas_call(
        _affine_kernel,
        out_shape=jax.ShapeDtypeStruct((b, c, hw), jnp.float32),
        grid=(b // nb,),
        in_specs=[
            pl.BlockSpec((nb, c, hw), lambda i: (i, 0, 0)),
            pl.BlockSpec((c, 1), lambda i: (0, 0)),
            pl.BlockSpec((c, 1), lambda i: (0, 0)),
        ],
        out_specs=pl.BlockSpec((nb, c, hw), lambda i: (i, 0, 0)),
        compiler_params=_CP,
    )(y, scale[:, None], shift[:, None])


def kernel(w_h2h, w_h2l, w_l2h, w_l2l, gamma_h, beta_h, gamma_l, beta_l,
           x_h, x_l, eps=1e-5):
    b, cih, h, w = x_h.shape
    _, cil, hl, wl = x_l.shape
    coh = w_h2h.shape[0]
    col = w_l2l.shape[0]
    assert h & (h - 1) == 0 and w & (w - 1) == 0, "spatial dims must be pow2"
    hw, hwl = h * w, hl * wl
    mh, ml = b * hw, b * hwl

    q = jnp.arange(hw)
    p_of_q = (q // (2 * w)) * wl + (q % w) // 2
    sel = p_of_q[:, None] == jnp.arange(hwl)[None, :]
    pool_mat = jnp.where(sel, 0.25, 0.0).astype(jnp.bfloat16)
    up_mat = jnp.where(sel, 1.0, 0.0).astype(jnp.bfloat16).T

    nb = 8 if b % 8 == 0 else 1
    kfn = functools.partial(_fused_kernel, h=h, w=w, hl=hl, wl=wl, nb=nb)
    y_hi, y_lo, p_hi, p_lo = pl.pallas_call(
        kfn,
        out_shape=(jax.ShapeDtypeStruct((b, coh, hw), jnp.bfloat16),
                   jax.ShapeDtypeStruct((b, col, hwl), jnp.bfloat16),
                   jax.ShapeDtypeStruct((b, coh, 2), jnp.float32),
                   jax.ShapeDtypeStruct((b, col, 2), jnp.float32)),
        grid=(b // nb,),
        in_specs=[
            pl.BlockSpec((9, coh, cih), lambda i: (0, 0, 0)),
            pl.BlockSpec((9, col, cih), lambda i: (0, 0, 0)),
            pl.BlockSpec((9, col, cil), lambda i: (0, 0, 0)),
            pl.BlockSpec((9, coh, cil), lambda i: (0, 0, 0)),
            pl.BlockSpec((hw, hwl), lambda i: (0, 0)),
            pl.BlockSpec((hwl, hw), lambda i: (0, 0)),
            pl.BlockSpec((nb, cih, hw), lambda i: (i, 0, 0)),
            pl.BlockSpec((nb, cil, hwl), lambda i: (i, 0, 0)),
        ],
        out_specs=(pl.BlockSpec((nb, coh, hw), lambda i: (i, 0, 0)),
                   pl.BlockSpec((nb, col, hwl), lambda i: (i, 0, 0)),
                   pl.BlockSpec((nb, coh, 2), lambda i: (i, 0, 0)),
                   pl.BlockSpec((nb, col, 2), lambda i: (i, 0, 0))),
        compiler_params=_CP,
    )(_taps(w_h2h), _taps(w_h2l), _taps(w_l2l), _taps(w_l2h),
      pool_mat, up_mat, x_h.reshape(b, cih, hw), x_l.reshape(b, cil, hwl))

    out_h = _bn_apply(y_hi, p_hi, mh, gamma_h, beta_h, eps)
    out_l = _bn_apply(y_lo, p_lo, ml, gamma_l, beta_l, eps)
    return out_h.reshape(b, coh, h, w), out_l.reshape(b, col, hl, wl)
